# Initial kernel scaffold; baseline (speedup 1.0000x reference)
import jax
import jax.numpy as jnp
from jax.experimental import pallas as pl


def kernel(x, token_table):
    raise NotImplementedError("write your pallas kernel here")


# speedup vs baseline: 5.1690x; 5.1690x over previous
import functools

import jax
import jax.numpy as jnp
from jax import lax
from jax.experimental import pallas as pl
from jax.experimental.pallas import tpu as pltpu
from jax.experimental.pallas import tpu_sc as plsc

NC = 2
NS = 16
NW = NC * NS

VOCAB = 100000
HIDDEN = 128
N_ROWS = 4096 * 200
ROWS_PER_W = N_ROWS // NW
CHUNK = 128
N_CHUNKS = ROWS_PER_W // CHUNK


def _emb_body(table_hbm, idx_hbm, out_hbm, idx_v, rows_v, sem):
    wid = lax.axis_index("s") * NC + lax.axis_index("c")
    base = wid * ROWS_PER_W

    def chunk(i, carry):
        off = base + i * CHUNK
        pltpu.sync_copy(idx_hbm.at[pl.ds(off, CHUNK)], idx_v)
        pltpu.async_copy(table_hbm.at[idx_v], rows_v, sem).wait()
        pltpu.sync_copy(rows_v, out_hbm.at[pl.ds(off, CHUNK)])
        return carry

    lax.fori_loop(0, N_CHUNKS, chunk, 0)


@jax.jit
def _embed(x_flat, token_table):
    k = functools.partial(
        pl.kernel,
        mesh=plsc.VectorSubcoreMesh(core_axis_name="c", subcore_axis_name="s"),
        out_type=jax.ShapeDtypeStruct((N_ROWS, HIDDEN), jnp.float32),
        scratch_types=[
            pltpu.VMEM((CHUNK,), jnp.int32),
            pltpu.VMEM((CHUNK, HIDDEN), jnp.float32),
            pltpu.SemaphoreType.DMA,
        ],
    )(_emb_body)
    return k(token_table, x_flat)


def kernel(x, token_table):
    b, s = x.shape
    out = _embed(x.reshape(-1), token_table)
    return out.reshape(b, s, HIDDEN)


# speedup vs baseline: 9.1899x; 1.7779x over previous
import functools

import jax
import jax.numpy as jnp
from jax import lax
from jax.experimental import pallas as pl
from jax.experimental.pallas import tpu as pltpu
from jax.experimental.pallas import tpu_sc as plsc

NC = 2
NS = 16
NW = NC * NS

HIDDEN = 128
N_ROWS = 4096 * 200
ROWS_PER_W = N_ROWS // NW
CHUNK = 128
N_CHUNKS = ROWS_PER_W // CHUNK
D = 4


def _emb_body(table_hbm, idx_hbm, out_hbm, idx_v, rows_v, *sems):
    gsem = sems[:D]
    osem = sems[D:]
    wid = lax.axis_index("s") * NC + lax.axis_index("c")
    base = wid * ROWS_PER_W
    pltpu.sync_copy(idx_hbm.at[pl.ds(base, ROWS_PER_W)], idx_v)

    def fire_g(i, slot):
        pltpu.async_copy(
            table_hbm.at[idx_v.at[pl.ds(i * CHUNK, CHUNK)]],
            rows_v.at[pl.ds(slot * CHUNK, CHUNK)],
            gsem[slot],
        )

    def wait_g(slot):
        pltpu.make_async_copy(
            table_hbm.at[idx_v.at[pl.ds(0, CHUNK)]],
            rows_v.at[pl.ds(slot * CHUNK, CHUNK)],
            gsem[slot],
        ).wait()

    def fire_o(i, slot):
        pltpu.async_copy(
            rows_v.at[pl.ds(slot * CHUNK, CHUNK)],
            out_hbm.at[pl.ds(base + i * CHUNK, CHUNK)],
            osem[slot],
        )

    def wait_o(slot):
        pltpu.make_async_copy(
            rows_v.at[pl.ds(slot * CHUNK, CHUNK)],
            out_hbm.at[pl.ds(base, CHUNK)],
            osem[slot],
        ).wait()

    for b in range(D):
        fire_g(b, b)
    for b in range(D - 1):
        wait_g(b)
        fire_o(b, b)

    def body(j, carry):
        i0 = D + j * D
        for b in range(D):
            i = i0 + b
            wait_o(b)
            fire_g(i, b)
            q = (b - 1) % D
            wait_g(q)
            fire_o(i - 1, q)
        return carry

    lax.fori_loop(0, (N_CHUNKS - D) // D, body, 0)

    last = (N_CHUNKS - 1) % D
    wait_g(last)
    fire_o(N_CHUNKS - 1, last)
    for b in range(D):
        wait_o(b)


@jax.jit
def _embed(x_flat, token_table):
    k = functools.partial(
        pl.kernel,
        mesh=plsc.VectorSubcoreMesh(core_axis_name="c", subcore_axis_name="s"),
        out_type=jax.ShapeDtypeStruct((N_ROWS, HIDDEN), jnp.float32),
        scratch_types=[
            pltpu.VMEM((ROWS_PER_W,), jnp.int32),
            pltpu.VMEM((D * CHUNK, HIDDEN), jnp.float32),
        ]
        + [pltpu.SemaphoreType.DMA] * (2 * D),
    )(_emb_body)
    return k(token_table, x_flat)


def kernel(x, token_table):
    b, s = x.shape
    out = _embed(x.reshape(-1), token_table)
    return out.reshape(b, s, HIDDEN)


# speedup vs baseline: 9.2158x; 1.0028x over previous
import functools

import jax
import jax.numpy as jnp
from jax import lax
from jax.experimental import pallas as pl
from jax.experimental.pallas import tpu as pltpu
from jax.experimental.pallas import tpu_sc as plsc

NC = 2
NS = 16
NW = NC * NS

HIDDEN = 128
N_ROWS = 4096 * 200
ROWS_PER_W = N_ROWS // NW
CHUNK = 128
N_CHUNKS = ROWS_PER_W // CHUNK
D = 5


def _emb_body(table_hbm, idx_hbm, out_hbm, idx_v, rows_v, *sems):
    gsem = sems[:D]
    osem = sems[D:]
    wid = lax.axis_index("s") * NC + lax.axis_index("c")
    base = wid * ROWS_PER_W
    pltpu.sync_copy(idx_hbm.at[pl.ds(base, ROWS_PER_W)], idx_v)

    def fire_g(i, slot):
        pltpu.async_copy(
            table_hbm.at[idx_v.at[pl.ds(i * CHUNK, CHUNK)]],
            rows_v.at[pl.ds(slot * CHUNK, CHUNK)],
            gsem[slot],
        )

    def wait_g(slot):
        pltpu.make_async_copy(
            table_hbm.at[idx_v.at[pl.ds(0, CHUNK)]],
            rows_v.at[pl.ds(slot * CHUNK, CHUNK)],
            gsem[slot],
        ).wait()

    def fire_o(i, slot):
        pltpu.async_copy(
            rows_v.at[pl.ds(slot * CHUNK, CHUNK)],
            out_hbm.at[pl.ds(base + i * CHUNK, CHUNK)],
            osem[slot],
        )

    def wait_o(slot):
        pltpu.make_async_copy(
            rows_v.at[pl.ds(slot * CHUNK, CHUNK)],
            out_hbm.at[pl.ds(base, CHUNK)],
            osem[slot],
        ).wait()

    for b in range(D):
        fire_g(b, b)
    for b in range(D - 1):
        wait_g(b)
        fire_o(b, b)

    def body(j, carry):
        i0 = D + j * D
        for b in range(D):
            i = i0 + b
            wait_o(b)
            fire_g(i, b)
            q = (b - 1) % D
            wait_g(q)
            fire_o(i - 1, q)
        return carry

    lax.fori_loop(0, (N_CHUNKS - D) // D, body, 0)

    last = (N_CHUNKS - 1) % D
    wait_g(last)
    fire_o(N_CHUNKS - 1, last)
    for b in range(D):
        wait_o(b)


@jax.jit
def _embed(x_flat, token_table):
    k = functools.partial(
        pl.kernel,
        mesh=plsc.VectorSubcoreMesh(core_axis_name="c", subcore_axis_name="s"),
        out_type=jax.ShapeDtypeStruct((N_ROWS, HIDDEN), jnp.float32),
        scratch_types=[
            pltpu.VMEM((ROWS_PER_W,), jnp.int32),
            pltpu.VMEM((D * CHUNK, HIDDEN), jnp.float32),
        ]
        + [pltpu.SemaphoreType.DMA] * (2 * D),
    )(_emb_body)
    return k(token_table, x_flat)


def kernel(x, token_table):
    b, s = x.shape
    out = _embed(x.reshape(-1), token_table)
    return out.reshape(b, s, HIDDEN)
